# baseline (speedup 1.0000x reference)
import functools

import jax
import jax.numpy as jnp
from jax import lax
from jax.experimental import pallas as pl
from jax.experimental.pallas import tpu as pltpu
from jax.experimental.pallas import tpu_sc as plsc

N_USERS = 25000
N_ITEMS = 25000
N = N_USERS + N_ITEMS
D = 64
N_LAYERS = 3
E = 800000

NC = 2
NS = 16
NQ = 4
QN = N // NQ
AR = 12560
ZCH = AR // NS
WB = 781
K = 512
CHUNKS = 50
EPW = K * CHUNKS
BCAP = CHUNKS + 2


_DIAG_NO_SCATTER = True


def _select_lane(vec, lane):
    idx = lax.iota(jnp.int32, 16)
    return jnp.sum(jnp.where(idx == lane, vec, 0))


def _part_body(ed_hbm, bkt_hbm, cnt_hbm,
               in0, in1, pend, cbuf, sem_e, sem_f):
    c = lax.axis_index("c")
    s = lax.axis_index("s")
    w = c * NS + s
    cbase = w * CHUNKS

    def _fetch(i, buf):
        return pltpu.async_copy(ed_hbm.at[cbase + i], buf, sem_e)

    def _flush(q, chn):
        for f in range(3):
            pltpu.async_copy(pend.at[q, f, pl.ds(0, K)],
                             bkt_hbm.at[q, w, chn, f], sem_f).wait()

    def _compact(buf, carry):
        def _group(g, carry):
            sl = pl.ds(g * 16, 16)
            rvec = buf[0, sl]
            cvec = buf[1, sl]
            vvec = buf[2, sl]
            new = []
            for q in range(NQ):
                wq, chq = carry[2 * q], carry[2 * q + 1]
                mq = (rvec >= q * QN) & (rvec < (q + 1) * QN)
                loc = rvec - q * QN
                plsc.store_compressed(pend.at[q, 0, pl.ds(wq, 16)], loc,
                                      mask=mq)
                plsc.store_compressed(pend.at[q, 1, pl.ds(wq, 16)], cvec,
                                      mask=mq)
                plsc.store_compressed(pend.at[q, 2, pl.ds(wq, 16)], vvec,
                                      mask=mq)
                pc = plsc.all_reduce_population_count(mq)
                wq = wq + pc[0]

                def _fire(op):
                    wq, chq = op
                    _flush(q, chq)
                    for f in range(3):
                        ov = pend[q, f, pl.ds(K, 16)]
                        pend[q, f, pl.ds(0, 16)] = ov
                    return wq - K, chq + 1

                wq, chq = lax.cond(wq >= K, _fire, lambda op: op, (wq, chq))
                new.extend([wq, chq])
            return tuple(new)
        return lax.fori_loop(0, K // 16, _group, carry)

    _fetch(0, in0)
    _fetch(1, in1)
    carry = (jnp.int32(0),) * (2 * NQ)

    def _ppair(j, carry):
        pltpu.make_async_copy(ed_hbm.at[0], in0, sem_e).wait()
        carry = _compact(in0, carry)

        @pl.when(2 * j + 2 < CHUNKS)
        def _():
            _fetch(2 * j + 2, in0)

        pltpu.make_async_copy(ed_hbm.at[0], in1, sem_e).wait()
        carry = _compact(in1, carry)

        @pl.when(2 * j + 3 < CHUNKS)
        def _():
            _fetch(2 * j + 3, in1)
        return carry

    carry = lax.fori_loop(0, CHUNKS // 2, _ppair, carry)

    lanes = lax.iota(jnp.int32, 16)
    npairs = []
    for q in range(NQ):
        wq, chq = carry[2 * q], carry[2 * q + 1]

        def _pad(g, wq=wq, q=q):
            sl = pl.ds(g * 16, 16)
            pos = lanes + g * 16
            keep = pos < wq
            pend[q, 0, sl] = jnp.where(keep, pend[q, 0, sl], QN)
            pend[q, 1, sl] = jnp.where(keep, pend[q, 1, sl], 0)
            pend[q, 2, sl] = jnp.where(keep, pend[q, 2, sl], 0)

        for g in range(K // 16 + 1):
            _pad(g)
        _flush(q, chq)
        chq = chq + 1

        def _extra(chq, q=q):
            for g in range(K // 16 + 1):
                sl = pl.ds(g * 16, 16)
                pend[q, 0, sl] = jnp.full((16,), QN, jnp.int32)
                pend[q, 1, sl] = jnp.zeros((16,), jnp.int32)
                pend[q, 2, sl] = jnp.zeros((16,), jnp.int32)
            _flush(q, chq)
            return chq + 1

        chq = lax.cond(chq % 2 == 1, _extra, lambda x: x, chq)
        npairs.append(chq // 2)

    vec = jnp.zeros((16,), jnp.int32)
    for q in range(NQ):
        vec = jnp.where(lanes == q, npairs[q], vec)
    cbuf[pl.ds(0, 16)] = vec
    pltpu.sync_copy(cbuf, cnt_hbm.at[pl.ds(w * 16, 16)])


_partition = functools.partial(
    pl.kernel,
    out_type=(
        jax.ShapeDtypeStruct((NQ, NC * NS, BCAP, 3, K), jnp.int32),
        jax.ShapeDtypeStruct((NC * NS * 16,), jnp.int32),
    ),
    mesh=plsc.VectorSubcoreMesh(core_axis_name="c", subcore_axis_name="s"),
    compiler_params=pltpu.CompilerParams(
        use_tc_tiling_on_sc=False, needs_layout_passes=False),
    scratch_types=[
        pltpu.VMEM((3, K), jnp.int32),
        pltpu.VMEM((3, K), jnp.int32),
        pltpu.VMEM((NQ, 3, K + 16), jnp.int32),
        pltpu.VMEM((16,), jnp.int32),
        pltpu.SemaphoreType.DMA,
        pltpu.SemaphoreType.DMA,
    ],
)(_part_body)


def _layer_body(x_hbm, bkt_hbm, cnt_hbm, y_hbm,
                ebuf0, ebuf1, loc0, loc1, rows0, rows1, cntv, acc,
                sem_e0, sem_e1, sem_g0, sem_g1, sem_s0, sem_s1):
    c = lax.axis_index("c")
    s = lax.axis_index("s")
    banks = ((ebuf0, loc0, rows0, sem_e0, sem_g0, sem_s0),
             (ebuf1, loc1, rows1, sem_e1, sem_g1, sem_s1))

    pltpu.sync_copy(cnt_hbm, cntv)

    for p in range(2):
        q = c * 2 + p
        base_row = q * QN

        def _z(i, _):
            z = jnp.zeros((16,), jnp.float32)
            for d in range(D // 16):
                rows0[i, pl.ds(d * 16, 16)] = z
            return 0
        lax.fori_loop(0, K, _z, 0)
        zbase = s * ZCH
        pltpu.sync_copy(rows0.at[pl.ds(0, K)], acc.at[pl.ds(zbase, K)])
        pltpu.sync_copy(rows0.at[pl.ds(0, ZCH - K)],
                        acc.at[pl.ds(zbase + K, ZCH - K)])
        plsc.subcore_barrier()

        def _drain_scatter(bank):
            if _DIAG_NO_SCATTER:
                return
            _, loc, rows_v, _, _, sem_s = bank
            pltpu.make_async_copy(rows_v, acc.at[loc], sem_s).wait()

        def _process(bank):
            ebuf, loc, rows_v, _, _, sem_s = bank

            def _cl(j, _):
                sl = pl.ds(j * 16, 16)
                loc[sl] = ebuf[0, sl]
                vv = plsc.bitcast(ebuf[2, sl], jnp.float32)
                for l in range(16):
                    bv = jnp.broadcast_to(vv[l], (16,))
                    e = j * 16 + l
                    for d in range(D // 16):
                        rsl = pl.ds(d * 16, 16)
                        rows_v[e, rsl] = rows_v[e, rsl] * bv
                return 0
            lax.fori_loop(0, K // 16, _cl, 0)
            if _DIAG_NO_SCATTER:
                pass
            else:
                pltpu.async_copy(rows_v, acc.at[loc], sem_s, add=True)

        for cc in range(NC):
            gsl = pl.ds((cc * NS + s) * 16, 16)
            cvec = cntv[gsl]
            npairs = _select_lane(cvec, q)
            region = cc * NS + s

            def _pair(j, _):
                descs = []
                for b in range(2):
                    bank = banks[b]
                    ebuf, _, rows_v, sem_e, sem_g, _ = bank

                    @pl.when(j > 0)
                    def _():
                        _drain_scatter(bank)

                    pltpu.async_copy(
                        bkt_hbm.at[q, region, 2 * j + b], ebuf, sem_e).wait()
                    descs.append(pltpu.async_copy(
                        x_hbm.at[ebuf.at[1]], rows_v, sem_g))
                for b in range(2):
                    descs[b].wait()
                    _process(banks[b])
                return 0

            lax.fori_loop(0, npairs, _pair, 0)

            @pl.when(npairs > 0)
            def _():
                _drain_scatter(banks[0])
                _drain_scatter(banks[1])

        plsc.subcore_barrier()

        wb = s * WB
        pltpu.sync_copy(acc.at[pl.ds(wb, WB)],
                        y_hbm.at[pl.ds(base_row + wb, WB)])

        @pl.when(s == 0)
        def _():
            pltpu.sync_copy(acc.at[pl.ds(NS * WB, QN - NS * WB)],
                            y_hbm.at[pl.ds(base_row + NS * WB, QN - NS * WB)])

        plsc.subcore_barrier()


_layer = functools.partial(
    pl.kernel,
    out_type=jax.ShapeDtypeStruct((N, D), jnp.float32),
    mesh=plsc.VectorSubcoreMesh(core_axis_name="c", subcore_axis_name="s"),
    compiler_params=pltpu.CompilerParams(
        use_tc_tiling_on_sc=False, needs_layout_passes=False),
    scratch_types=[
        pltpu.VMEM((3, K), jnp.int32),
        pltpu.VMEM((3, K), jnp.int32),
        pltpu.VMEM((K,), jnp.int32),
        pltpu.VMEM((K,), jnp.int32),
        pltpu.VMEM((K, D), jnp.float32),
        pltpu.VMEM((K, D), jnp.float32),
        pltpu.VMEM((NC * NS * 16,), jnp.int32),
        pltpu.VMEM_SHARED((AR, D), jnp.float32),
        pltpu.SemaphoreType.DMA,
        pltpu.SemaphoreType.DMA,
        pltpu.SemaphoreType.DMA,
        pltpu.SemaphoreType.DMA,
        pltpu.SemaphoreType.DMA,
        pltpu.SemaphoreType.DMA,
    ],
)(_layer_body)


def _mean_body(x0, x1, x2, x3, o):
    o[...] = (x0[...] + x1[...] + x2[...] + x3[...]) * 0.25


def _mean(x0, x1, x2, x3):
    blk = 400
    grid = N // blk
    spec = pl.BlockSpec((blk, D), lambda i: (i, 0))
    return pl.pallas_call(
        _mean_body,
        grid=(grid,),
        in_specs=[spec] * 4,
        out_specs=spec,
        out_shape=jax.ShapeDtypeStruct((N, D), jnp.float32),
    )(x0, x1, x2, x3)


def kernel(adj_indices, adj_values, user_emb, item_emb):
    row = adj_indices[0].astype(jnp.int32)
    col = adj_indices[1].astype(jnp.int32)
    val = adj_values.astype(jnp.float32)

    pad = NC * NS * EPW - E
    row = jnp.concatenate([row, jnp.full((pad,), N, jnp.int32)])
    col = jnp.concatenate([col, jnp.zeros((pad,), jnp.int32)])
    vbits = lax.bitcast_convert_type(
        jnp.concatenate([val, jnp.zeros((pad,), jnp.float32)]), jnp.int32)
    edges = jnp.stack(
        [row.reshape(-1, K), col.reshape(-1, K), vbits.reshape(-1, K)],
        axis=1)

    bkt, cnt = _partition(edges)

    x0 = jnp.concatenate([user_emb, item_emb], axis=0)
    xs = [x0]
    x = x0
    for _ in range(N_LAYERS):
        x = _layer(x, bkt, cnt)
        xs.append(x)

    out = _mean(*xs)
    return (out[:N_USERS], out[N_USERS:])


# speedup vs baseline: 1.0014x; 1.0014x over previous
import functools

import jax
import jax.numpy as jnp
from jax import lax
from jax.experimental import pallas as pl
from jax.experimental.pallas import tpu as pltpu
from jax.experimental.pallas import tpu_sc as plsc

N_USERS = 25000
N_ITEMS = 25000
N = N_USERS + N_ITEMS
D = 64
N_LAYERS = 3
E = 800000

NC = 2
NS = 16
NQ = 4
QN = N // NQ
AR = 12560
ZCH = AR // NS
WB = 781
K = 512
CHUNKS = 50
EPW = K * CHUNKS
BCAP = CHUNKS + 2


def _select_lane(vec, lane):
    idx = lax.iota(jnp.int32, 16)
    return jnp.sum(jnp.where(idx == lane, vec, 0))


def _part_body(ed_hbm, bkt_hbm, cnt_hbm,
               in0, in1, pend, cbuf, sem_e, sem_f):
    c = lax.axis_index("c")
    s = lax.axis_index("s")
    w = c * NS + s
    cbase = w * CHUNKS

    def _fetch(i, buf):
        return pltpu.async_copy(ed_hbm.at[cbase + i], buf, sem_e)

    def _flush(q, chn):
        for f in range(3):
            pltpu.async_copy(pend.at[q, f, pl.ds(0, K)],
                             bkt_hbm.at[q, w, chn, f], sem_f).wait()

    def _compact(buf, carry):
        def _group(g, carry):
            sl = pl.ds(g * 16, 16)
            rvec = buf[0, sl]
            cvec = buf[1, sl]
            vvec = buf[2, sl]
            new = []
            for q in range(NQ):
                wq, chq = carry[2 * q], carry[2 * q + 1]
                mq = (rvec >= q * QN) & (rvec < (q + 1) * QN)
                loc = rvec - q * QN
                plsc.store_compressed(pend.at[q, 0, pl.ds(wq, 16)], loc,
                                      mask=mq)
                plsc.store_compressed(pend.at[q, 1, pl.ds(wq, 16)], cvec,
                                      mask=mq)
                plsc.store_compressed(pend.at[q, 2, pl.ds(wq, 16)], vvec,
                                      mask=mq)
                pc = plsc.all_reduce_population_count(mq)
                wq = wq + pc[0]

                def _fire(op):
                    wq, chq = op
                    _flush(q, chq)
                    for f in range(3):
                        ov = pend[q, f, pl.ds(K, 16)]
                        pend[q, f, pl.ds(0, 16)] = ov
                    return wq - K, chq + 1

                wq, chq = lax.cond(wq >= K, _fire, lambda op: op, (wq, chq))
                new.extend([wq, chq])
            return tuple(new)
        return lax.fori_loop(0, K // 16, _group, carry)

    _fetch(0, in0)
    _fetch(1, in1)
    carry = (jnp.int32(0),) * (2 * NQ)

    def _ppair(j, carry):
        pltpu.make_async_copy(ed_hbm.at[0], in0, sem_e).wait()
        carry = _compact(in0, carry)

        @pl.when(2 * j + 2 < CHUNKS)
        def _():
            _fetch(2 * j + 2, in0)

        pltpu.make_async_copy(ed_hbm.at[0], in1, sem_e).wait()
        carry = _compact(in1, carry)

        @pl.when(2 * j + 3 < CHUNKS)
        def _():
            _fetch(2 * j + 3, in1)
        return carry

    carry = lax.fori_loop(0, CHUNKS // 2, _ppair, carry)

    lanes = lax.iota(jnp.int32, 16)
    npairs = []
    for q in range(NQ):
        wq, chq = carry[2 * q], carry[2 * q + 1]

        def _pad(g, wq=wq, q=q):
            sl = pl.ds(g * 16, 16)
            pos = lanes + g * 16
            keep = pos < wq
            pend[q, 0, sl] = jnp.where(keep, pend[q, 0, sl], QN)
            pend[q, 1, sl] = jnp.where(keep, pend[q, 1, sl], 0)
            pend[q, 2, sl] = jnp.where(keep, pend[q, 2, sl], 0)

        for g in range(K // 16 + 1):
            _pad(g)
        _flush(q, chq)
        chq = chq + 1

        def _extra(chq, q=q):
            for g in range(K // 16 + 1):
                sl = pl.ds(g * 16, 16)
                pend[q, 0, sl] = jnp.full((16,), QN, jnp.int32)
                pend[q, 1, sl] = jnp.zeros((16,), jnp.int32)
                pend[q, 2, sl] = jnp.zeros((16,), jnp.int32)
            _flush(q, chq)
            return chq + 1

        chq = lax.cond(chq % 2 == 1, _extra, lambda x: x, chq)
        npairs.append(chq // 2)

    vec = jnp.zeros((16,), jnp.int32)
    for q in range(NQ):
        vec = jnp.where(lanes == q, npairs[q], vec)
    cbuf[pl.ds(0, 16)] = vec
    pltpu.sync_copy(cbuf, cnt_hbm.at[pl.ds(w * 16, 16)])


_partition = functools.partial(
    pl.kernel,
    out_type=(
        jax.ShapeDtypeStruct((NQ, NC * NS, BCAP, 3, K), jnp.int32),
        jax.ShapeDtypeStruct((NC * NS * 16,), jnp.int32),
    ),
    mesh=plsc.VectorSubcoreMesh(core_axis_name="c", subcore_axis_name="s"),
    compiler_params=pltpu.CompilerParams(
        use_tc_tiling_on_sc=False, needs_layout_passes=False),
    scratch_types=[
        pltpu.VMEM((3, K), jnp.int32),
        pltpu.VMEM((3, K), jnp.int32),
        pltpu.VMEM((NQ, 3, K + 16), jnp.int32),
        pltpu.VMEM((16,), jnp.int32),
        pltpu.SemaphoreType.DMA,
        pltpu.SemaphoreType.DMA,
    ],
)(_part_body)


def _layer_body(x_hbm, bkt_hbm, cnt_hbm, y_hbm,
                ebuf0, ebuf1, loc0, loc1, rows0, rows1, cntv, acc,
                sem_e0, sem_e1, sem_g0, sem_g1, sem_s0, sem_s1):
    c = lax.axis_index("c")
    s = lax.axis_index("s")
    banks = ((ebuf0, loc0, rows0, sem_e0, sem_g0, sem_s0),
             (ebuf1, loc1, rows1, sem_e1, sem_g1, sem_s1))

    pltpu.sync_copy(cnt_hbm, cntv)

    for p in range(2):
        q = c * 2 + p
        base_row = q * QN

        def _z(i, _):
            z = jnp.zeros((16,), jnp.float32)
            for d in range(D // 16):
                rows0[i, pl.ds(d * 16, 16)] = z
            return 0
        lax.fori_loop(0, K, _z, 0)
        zbase = s * ZCH
        pltpu.sync_copy(rows0.at[pl.ds(0, K)], acc.at[pl.ds(zbase, K)])
        pltpu.sync_copy(rows0.at[pl.ds(0, ZCH - K)],
                        acc.at[pl.ds(zbase + K, ZCH - K)])
        plsc.subcore_barrier()

        def _drain_scatter(bank):
            _, loc, rows_v, _, _, sem_s = bank
            pltpu.make_async_copy(rows_v, acc.at[loc], sem_s).wait()

        def _process(bank):
            ebuf, loc, rows_v, _, _, sem_s = bank

            def _cl(j, _):
                sl = pl.ds(j * 16, 16)
                loc[sl] = ebuf[0, sl]
                vv = plsc.bitcast(ebuf[2, sl], jnp.float32)
                for l in range(16):
                    bv = jnp.broadcast_to(vv[l], (16,))
                    e = j * 16 + l
                    for d in range(D // 16):
                        rsl = pl.ds(d * 16, 16)
                        rows_v[e, rsl] = rows_v[e, rsl] * bv
                return 0
            lax.fori_loop(0, K // 16, _cl, 0)
            pltpu.async_copy(rows_v, acc.at[loc], sem_s, add=True)

        for cc in range(NC):
            gsl = pl.ds((cc * NS + s) * 16, 16)
            cvec = cntv[gsl]
            npairs = _select_lane(cvec, q)
            region = cc * NS + s

            def _pair(j, _):
                @pl.when(j < npairs)
                def _():
                    descs = []
                    for b in range(2):
                        bank = banks[b]
                        ebuf, _, rows_v, sem_e, sem_g, _ = bank

                        @pl.when(j > 0)
                        def _():
                            _drain_scatter(bank)

                        pltpu.async_copy(
                            bkt_hbm.at[q, region, 2 * j + b], ebuf,
                            sem_e).wait()
                        descs.append(pltpu.async_copy(
                            x_hbm.at[ebuf.at[1]], rows_v, sem_g))
                    for b in range(2):
                        descs[b].wait()
                        _process(banks[b])
                return 0

            lax.fori_loop(0, BCAP // 2, _pair, 0)

            @pl.when(npairs > 0)
            def _():
                _drain_scatter(banks[0])
                _drain_scatter(banks[1])

        plsc.subcore_barrier()

        wb = s * WB
        pltpu.sync_copy(acc.at[pl.ds(wb, WB)],
                        y_hbm.at[pl.ds(base_row + wb, WB)])

        @pl.when(s == 0)
        def _():
            pltpu.sync_copy(acc.at[pl.ds(NS * WB, QN - NS * WB)],
                            y_hbm.at[pl.ds(base_row + NS * WB, QN - NS * WB)])

        plsc.subcore_barrier()


_layer = functools.partial(
    pl.kernel,
    out_type=jax.ShapeDtypeStruct((N, D), jnp.float32),
    mesh=plsc.VectorSubcoreMesh(core_axis_name="c", subcore_axis_name="s"),
    compiler_params=pltpu.CompilerParams(
        use_tc_tiling_on_sc=False, needs_layout_passes=False),
    scratch_types=[
        pltpu.VMEM((3, K), jnp.int32),
        pltpu.VMEM((3, K), jnp.int32),
        pltpu.VMEM((K,), jnp.int32),
        pltpu.VMEM((K,), jnp.int32),
        pltpu.VMEM((K, D), jnp.float32),
        pltpu.VMEM((K, D), jnp.float32),
        pltpu.VMEM((NC * NS * 16,), jnp.int32),
        pltpu.VMEM_SHARED((AR, D), jnp.float32),
        pltpu.SemaphoreType.DMA,
        pltpu.SemaphoreType.DMA,
        pltpu.SemaphoreType.DMA,
        pltpu.SemaphoreType.DMA,
        pltpu.SemaphoreType.DMA,
        pltpu.SemaphoreType.DMA,
    ],
)(_layer_body)


def _mean_body(x0, x1, x2, x3, o):
    o[...] = (x0[...] + x1[...] + x2[...] + x3[...]) * 0.25


def _mean(x0, x1, x2, x3):
    blk = 400
    grid = N // blk
    spec = pl.BlockSpec((blk, D), lambda i: (i, 0))
    return pl.pallas_call(
        _mean_body,
        grid=(grid,),
        in_specs=[spec] * 4,
        out_specs=spec,
        out_shape=jax.ShapeDtypeStruct((N, D), jnp.float32),
    )(x0, x1, x2, x3)


def kernel(adj_indices, adj_values, user_emb, item_emb):
    row = adj_indices[0].astype(jnp.int32)
    col = adj_indices[1].astype(jnp.int32)
    val = adj_values.astype(jnp.float32)

    pad = NC * NS * EPW - E
    row = jnp.concatenate([row, jnp.full((pad,), N, jnp.int32)])
    col = jnp.concatenate([col, jnp.zeros((pad,), jnp.int32)])
    vbits = lax.bitcast_convert_type(
        jnp.concatenate([val, jnp.zeros((pad,), jnp.float32)]), jnp.int32)
    edges = jnp.stack(
        [row.reshape(-1, K), col.reshape(-1, K), vbits.reshape(-1, K)],
        axis=1)

    bkt, cnt = _partition(edges)

    x0 = jnp.concatenate([user_emb, item_emb], axis=0)
    xs = [x0]
    x = x0
    for _ in range(N_LAYERS):
        x = _layer(x, bkt, cnt)
        xs.append(x)

    out = _mean(*xs)
    return (out[:N_USERS], out[N_USERS:])


# speedup vs baseline: 1.0020x; 1.0007x over previous
import functools

import jax
import jax.numpy as jnp
from jax import lax
from jax.experimental import pallas as pl
from jax.experimental.pallas import tpu as pltpu
from jax.experimental.pallas import tpu_sc as plsc

N_USERS = 25000
N_ITEMS = 25000
N = N_USERS + N_ITEMS
D = 64
N_LAYERS = 3
E = 800000

NC = 2
NS = 16
NQ = 4
QN = N // NQ
AR = 12560
ZCH = AR // NS
WB = 781
K = 512
CHUNKS = 50
EPW = K * CHUNKS
BCAP = CHUNKS + 2


def _select_lane(vec, lane):
    idx = lax.iota(jnp.int32, 16)
    return jnp.sum(jnp.where(idx == lane, vec, 0))


def _part_body(ed_hbm, bkt_hbm, cnt_hbm,
               in0, in1, pend, cbuf, sem_e, sem_f):
    c = lax.axis_index("c")
    s = lax.axis_index("s")
    w = c * NS + s
    cbase = w * CHUNKS

    def _fetch(i, buf):
        return pltpu.async_copy(ed_hbm.at[cbase + i], buf, sem_e)

    def _flush(q, chn):
        ci = (q * NC * NS + w) * BCAP + chn
        for f in range(3):
            pltpu.async_copy(pend.at[q, f, pl.ds(0, K)],
                             bkt_hbm.at[ci, f], sem_f).wait()

    def _compact(buf, carry):
        def _group(g, carry):
            sl = pl.ds(g * 16, 16)
            rvec = buf[0, sl]
            cvec = buf[1, sl]
            vvec = buf[2, sl]
            new = []
            for q in range(NQ):
                wq, chq = carry[2 * q], carry[2 * q + 1]
                mq = (rvec >= q * QN) & (rvec < (q + 1) * QN)
                loc = rvec - q * QN
                plsc.store_compressed(pend.at[q, 0, pl.ds(wq, 16)], loc,
                                      mask=mq)
                plsc.store_compressed(pend.at[q, 1, pl.ds(wq, 16)], cvec,
                                      mask=mq)
                plsc.store_compressed(pend.at[q, 2, pl.ds(wq, 16)], vvec,
                                      mask=mq)
                pc = plsc.all_reduce_population_count(mq)
                wq = wq + pc[0]

                def _fire(op):
                    wq, chq = op
                    _flush(q, chq)
                    for f in range(3):
                        ov = pend[q, f, pl.ds(K, 16)]
                        pend[q, f, pl.ds(0, 16)] = ov
                    return wq - K, chq + 1

                wq, chq = lax.cond(wq >= K, _fire, lambda op: op, (wq, chq))
                new.extend([wq, chq])
            return tuple(new)
        return lax.fori_loop(0, K // 16, _group, carry)

    _fetch(0, in0)
    _fetch(1, in1)
    carry = (jnp.int32(0),) * (2 * NQ)

    def _ppair(j, carry):
        pltpu.make_async_copy(ed_hbm.at[0], in0, sem_e).wait()
        carry = _compact(in0, carry)

        @pl.when(2 * j + 2 < CHUNKS)
        def _():
            _fetch(2 * j + 2, in0)

        pltpu.make_async_copy(ed_hbm.at[0], in1, sem_e).wait()
        carry = _compact(in1, carry)

        @pl.when(2 * j + 3 < CHUNKS)
        def _():
            _fetch(2 * j + 3, in1)
        return carry

    carry = lax.fori_loop(0, CHUNKS // 2, _ppair, carry)

    lanes = lax.iota(jnp.int32, 16)
    npairs = []
    for q in range(NQ):
        wq, chq = carry[2 * q], carry[2 * q + 1]

        def _pad(g, wq=wq, q=q):
            sl = pl.ds(g * 16, 16)
            pos = lanes + g * 16
            keep = pos < wq
            pend[q, 0, sl] = jnp.where(keep, pend[q, 0, sl], QN)
            pend[q, 1, sl] = jnp.where(keep, pend[q, 1, sl], 0)
            pend[q, 2, sl] = jnp.where(keep, pend[q, 2, sl], 0)

        for g in range(K // 16 + 1):
            _pad(g)
        _flush(q, chq)
        chq = chq + 1

        def _extra(chq, q=q):
            for g in range(K // 16 + 1):
                sl = pl.ds(g * 16, 16)
                pend[q, 0, sl] = jnp.full((16,), QN, jnp.int32)
                pend[q, 1, sl] = jnp.zeros((16,), jnp.int32)
                pend[q, 2, sl] = jnp.zeros((16,), jnp.int32)
            _flush(q, chq)
            return chq + 1

        chq = lax.cond(chq % 2 == 1, _extra, lambda x: x, chq)
        npairs.append(chq // 2)

    vec = jnp.zeros((16,), jnp.int32)
    for q in range(NQ):
        vec = jnp.where(lanes == q, npairs[q], vec)
    cbuf[pl.ds(0, 16)] = vec
    pltpu.sync_copy(cbuf, cnt_hbm.at[pl.ds(w * 16, 16)])


_partition = functools.partial(
    pl.kernel,
    out_type=(
        jax.ShapeDtypeStruct((NQ * NC * NS * BCAP, 3, K), jnp.int32),
        jax.ShapeDtypeStruct((NC * NS * 16,), jnp.int32),
    ),
    mesh=plsc.VectorSubcoreMesh(core_axis_name="c", subcore_axis_name="s"),
    compiler_params=pltpu.CompilerParams(
        use_tc_tiling_on_sc=False, needs_layout_passes=False),
    scratch_types=[
        pltpu.VMEM((3, K), jnp.int32),
        pltpu.VMEM((3, K), jnp.int32),
        pltpu.VMEM((NQ, 3, K + 16), jnp.int32),
        pltpu.VMEM((16,), jnp.int32),
        pltpu.SemaphoreType.DMA,
        pltpu.SemaphoreType.DMA,
    ],
)(_part_body)


def _layer_body(x_hbm, bkt_hbm, cnt_hbm, y_hbm,
                ebuf0, ebuf1, loc0, loc1, rows0, rows1, cntv, acc,
                sem_e0, sem_e1, sem_g0, sem_g1, sem_s0, sem_s1):
    c = lax.axis_index("c")
    s = lax.axis_index("s")
    banks = ((ebuf0, loc0, rows0, sem_e0, sem_g0, sem_s0),
             (ebuf1, loc1, rows1, sem_e1, sem_g1, sem_s1))

    pltpu.sync_copy(cnt_hbm, cntv)

    for p in range(2):
        q = c * 2 + p
        base_row = q * QN

        def _z(i, _):
            z = jnp.zeros((16,), jnp.float32)
            for d in range(D // 16):
                rows0[i, pl.ds(d * 16, 16)] = z
            return 0
        lax.fori_loop(0, K, _z, 0)
        zbase = s * ZCH
        pltpu.sync_copy(rows0.at[pl.ds(0, K)], acc.at[pl.ds(zbase, K)])
        pltpu.sync_copy(rows0.at[pl.ds(0, ZCH - K)],
                        acc.at[pl.ds(zbase + K, ZCH - K)])
        plsc.subcore_barrier()

        def _drain_scatter(bank):
            _, loc, rows_v, _, _, sem_s = bank
            pltpu.make_async_copy(rows_v, acc.at[loc], sem_s).wait()

        def _process(bank):
            ebuf, loc, rows_v, _, _, sem_s = bank

            def _cl(j, _):
                sl = pl.ds(j * 16, 16)
                loc[sl] = ebuf[0, sl]
                vv = plsc.bitcast(ebuf[2, sl], jnp.float32)
                for l in range(16):
                    bv = jnp.broadcast_to(vv[l], (16,))
                    e = j * 16 + l
                    for d in range(D // 16):
                        rsl = pl.ds(d * 16, 16)
                        rows_v[e, rsl] = rows_v[e, rsl] * bv
                return 0
            lax.fori_loop(0, K // 16, _cl, 0)
            pltpu.async_copy(rows_v, acc.at[loc], sem_s, add=True)

        for cc in range(NC):
            base_ci = (q * NC * NS + cc * NS + s) * BCAP
            gsl = pl.ds((cc * NS + s) * 16, 16)
            cvec = cntv[gsl]
            npairs = _select_lane(cvec, q)
            region = cc * NS + s

            def _pair(j, _):
                @pl.when(j < npairs)
                def _():
                    descs = []
                    for b in range(2):
                        bank = banks[b]
                        ebuf, _, rows_v, sem_e, sem_g, _ = bank

                        @pl.when(j > 0)
                        def _():
                            _drain_scatter(bank)

                        pltpu.async_copy(
                            bkt_hbm.at[base_ci + 2 * j + b], ebuf,
                            sem_e).wait()
                        descs.append(pltpu.async_copy(
                            x_hbm.at[ebuf.at[1]], rows_v, sem_g))
                    for b in range(2):
                        descs[b].wait()
                        _process(banks[b])
                return 0

            lax.fori_loop(0, BCAP // 2, _pair, 0)

            @pl.when(npairs > 0)
            def _():
                _drain_scatter(banks[0])
                _drain_scatter(banks[1])

        plsc.subcore_barrier()

        wb = s * WB
        pltpu.sync_copy(acc.at[pl.ds(wb, WB)],
                        y_hbm.at[pl.ds(base_row + wb, WB)])

        @pl.when(s == 0)
        def _():
            pltpu.sync_copy(acc.at[pl.ds(NS * WB, QN - NS * WB)],
                            y_hbm.at[pl.ds(base_row + NS * WB, QN - NS * WB)])

        plsc.subcore_barrier()


_layer = functools.partial(
    pl.kernel,
    out_type=jax.ShapeDtypeStruct((N, D), jnp.float32),
    mesh=plsc.VectorSubcoreMesh(core_axis_name="c", subcore_axis_name="s"),
    compiler_params=pltpu.CompilerParams(
        use_tc_tiling_on_sc=False, needs_layout_passes=False),
    scratch_types=[
        pltpu.VMEM((3, K), jnp.int32),
        pltpu.VMEM((3, K), jnp.int32),
        pltpu.VMEM((K,), jnp.int32),
        pltpu.VMEM((K,), jnp.int32),
        pltpu.VMEM((K, D), jnp.float32),
        pltpu.VMEM((K, D), jnp.float32),
        pltpu.VMEM((NC * NS * 16,), jnp.int32),
        pltpu.VMEM_SHARED((AR, D), jnp.float32),
        pltpu.SemaphoreType.DMA,
        pltpu.SemaphoreType.DMA,
        pltpu.SemaphoreType.DMA,
        pltpu.SemaphoreType.DMA,
        pltpu.SemaphoreType.DMA,
        pltpu.SemaphoreType.DMA,
    ],
)(_layer_body)


def _mean_body(x0, x1, x2, x3, o):
    o[...] = (x0[...] + x1[...] + x2[...] + x3[...]) * 0.25


def _mean(x0, x1, x2, x3):
    blk = 400
    grid = N // blk
    spec = pl.BlockSpec((blk, D), lambda i: (i, 0))
    return pl.pallas_call(
        _mean_body,
        grid=(grid,),
        in_specs=[spec] * 4,
        out_specs=spec,
        out_shape=jax.ShapeDtypeStruct((N, D), jnp.float32),
    )(x0, x1, x2, x3)


def kernel(adj_indices, adj_values, user_emb, item_emb):
    row = adj_indices[0].astype(jnp.int32)
    col = adj_indices[1].astype(jnp.int32)
    val = adj_values.astype(jnp.float32)

    pad = NC * NS * EPW - E
    row = jnp.concatenate([row, jnp.full((pad,), N, jnp.int32)])
    col = jnp.concatenate([col, jnp.zeros((pad,), jnp.int32)])
    vbits = lax.bitcast_convert_type(
        jnp.concatenate([val, jnp.zeros((pad,), jnp.float32)]), jnp.int32)
    edges = jnp.stack(
        [row.reshape(-1, K), col.reshape(-1, K), vbits.reshape(-1, K)],
        axis=1)

    bkt, cnt = _partition(edges)

    x0 = jnp.concatenate([user_emb, item_emb], axis=0)
    xs = [x0]
    x = x0
    for _ in range(N_LAYERS):
        x = _layer(x, bkt, cnt)
        xs.append(x)

    out = _mean(*xs)
    return (out[:N_USERS], out[N_USERS:])


# speedup vs baseline: 1.2602x; 1.2576x over previous
import functools

import jax
import jax.numpy as jnp
from jax import lax
from jax.experimental import pallas as pl
from jax.experimental.pallas import tpu as pltpu
from jax.experimental.pallas import tpu_sc as plsc

N_USERS = 25000
N_ITEMS = 25000
N = N_USERS + N_ITEMS
D = 64
N_LAYERS = 3
E = 800000

NC = 2
NS = 16
QN = N // 4
AR = 12560
ZCH = AR // NS
WB = 781
K = 512
CHUNKS = 98
EPT = K * CHUNKS
E_PAD = NS * EPT


def _zero_rows(rows_v):
    def _z(i, _):
        z = jnp.zeros((16,), jnp.float32)
        for d in range(D // 16):
            rows_v[i, pl.ds(d * 16, 16)] = z
        return 0
    lax.fori_loop(0, K, _z, 0)


def _layer_body(x_hbm, row_hbm, col_hbm, val_hbm, y_hbm,
                colv, rowlocv, valv, rows_v, acc, sem):
    c = lax.axis_index("c")
    s = lax.axis_index("s")
    ebase = s * EPT

    for p in range(2):
        base_row = (c * 2 + p) * QN

        _zero_rows(rows_v)
        zbase = s * ZCH
        pltpu.sync_copy(rows_v.at[pl.ds(0, K)], acc.at[pl.ds(zbase, K)])
        pltpu.sync_copy(rows_v.at[pl.ds(0, ZCH - K)],
                        acc.at[pl.ds(zbase + K, ZCH - K)])
        plsc.subcore_barrier()

        def _chunk(i, _):
            eb = ebase + i * K
            pltpu.sync_copy(col_hbm.at[pl.ds(eb, K)], colv)
            pltpu.async_copy(x_hbm.at[colv], rows_v, sem).wait()
            pltpu.sync_copy(row_hbm.at[pl.ds(eb, K)], colv)
            pltpu.sync_copy(val_hbm.at[pl.ds(eb, K)], valv)

            def _loc(j, _):
                r = colv[pl.ds(j * 16, 16)]
                loc = r - base_row
                ok = (loc >= 0) & (loc < QN)
                rowlocv[pl.ds(j * 16, 16)] = jnp.where(ok, loc, QN)
                return 0
            lax.fori_loop(0, K // 16, _loc, 0)

            def _scale(j, _):
                vv = valv[pl.ds(j * 16, 16)]
                for l in range(16):
                    bv = jnp.broadcast_to(vv[l], (16,))
                    e = j * 16 + l
                    for d in range(D // 16):
                        sl = pl.ds(d * 16, 16)
                        rows_v[e, sl] = rows_v[e, sl] * bv
                return 0
            lax.fori_loop(0, K // 16, _scale, 0)

            pltpu.sync_copy(rows_v, acc.at[rowlocv], add=True)
            return 0

        lax.fori_loop(0, CHUNKS, _chunk, 0)
        plsc.subcore_barrier()

        wb = s * WB
        pltpu.sync_copy(acc.at[pl.ds(wb, WB)],
                        y_hbm.at[pl.ds(base_row + wb, WB)])

        @pl.when(s == 0)
        def _():
            pltpu.sync_copy(acc.at[pl.ds(NS * WB, QN - NS * WB)],
                            y_hbm.at[pl.ds(base_row + NS * WB, QN - NS * WB)])

        plsc.subcore_barrier()


_layer = functools.partial(
    pl.kernel,
    out_type=jax.ShapeDtypeStruct((N, D), jnp.float32),
    mesh=plsc.VectorSubcoreMesh(core_axis_name="c", subcore_axis_name="s"),
    compiler_params=pltpu.CompilerParams(use_tc_tiling_on_sc=False),
    scratch_types=[
        pltpu.VMEM((K,), jnp.int32),
        pltpu.VMEM((K,), jnp.int32),
        pltpu.VMEM((K,), jnp.float32),
        pltpu.VMEM((K, D), jnp.float32),
        pltpu.VMEM_SHARED((AR, D), jnp.float32),
        pltpu.SemaphoreType.DMA,
    ],
)(_layer_body)


def _mean_body(x0, x1, x2, x3, o):
    o[...] = (x0[...] + x1[...] + x2[...] + x3[...]) * 0.25


def _mean(x0, x1, x2, x3):
    blk = 400
    grid = N // blk
    spec = pl.BlockSpec((blk, D), lambda i: (i, 0))
    return pl.pallas_call(
        _mean_body,
        grid=(grid,),
        in_specs=[spec] * 4,
        out_specs=spec,
        out_shape=jax.ShapeDtypeStruct((N, D), jnp.float32),
    )(x0, x1, x2, x3)


def kernel(adj_indices, adj_values, user_emb, item_emb):
    row = adj_indices[0].astype(jnp.int32)
    col = adj_indices[1].astype(jnp.int32)
    val = adj_values.astype(jnp.float32)

    pad = E_PAD - E
    row = jnp.concatenate([row, jnp.full((pad,), N, jnp.int32)])
    col = jnp.concatenate([col, jnp.zeros((pad,), jnp.int32)])
    val = jnp.concatenate([val, jnp.zeros((pad,), jnp.float32)])

    x0 = jnp.concatenate([user_emb, item_emb], axis=0)
    xs = [x0]
    x = x0
    for _ in range(N_LAYERS):
        x = _layer(x, row, col, val)
        xs.append(x)

    out = _mean(*xs)
    return (out[:N_USERS], out[N_USERS:])
